# Initial kernel scaffold; baseline (speedup 1.0000x reference)
import jax
import jax.numpy as jnp
from jax.experimental import pallas as pl


def kernel(weight, q_len, k_len):
    raise NotImplementedError("write your pallas kernel here")


# speedup vs baseline: 41.7108x; 41.7108x over previous
import functools
import jax
import jax.numpy as jnp
from jax import lax
from jax.experimental import pallas as pl
from jax.experimental.pallas import tpu as pltpu, tpu_sc as plsc

H = 16
Q = 2048
K = 2048
NB = 32
L = 16
RLEN = 4128
SLEN = 4112

_THRESH = (1, 2, 3, 4, 5, 6, 7, 8, 9, 10, 11, 12, 13, 14, 15, 16,
           19, 21, 24, 27, 31, 35, 40, 46, 52, 59, 67, 77, 87, 99, 113)

_WINDOW = 32


def _body(weight_hbm, off_hbm, out_hbm, w_v, off_v, rarr_v, shift_v, sem):
    c = lax.axis_index("c")
    s = lax.axis_index("s")
    h = s
    i0 = c * (Q // 2)

    pltpu.sync_copy(weight_hbm, w_v)
    pltpu.sync_copy(off_hbm, off_v)

    off_vec = off_v[...]
    h_vec = jnp.full((L,), h, dtype=jnp.int32)
    iota = lax.iota(jnp.int32, L)

    def build(u, _):
        v = u * L + iota
        n = jnp.maximum((Q - 1) - off_vec - v, 0)
        b = jnp.zeros((L,), jnp.int32)
        for t in _THRESH:
            b = b + (n >= t).astype(jnp.int32)
        vals = plsc.load_gather(w_v, [b, h_vec])
        rarr_v[pl.ds(u * L, L)] = vals
        return 0

    lax.fori_loop(0, RLEN // L, build, 0, unroll=False)

    def shift(u, _):
        base = u * L + iota
        for sh in range(8):
            shift_v[pl.ds(sh * SLEN + u * L, L)] = plsc.load_gather(rarr_v, [base + sh])
        return 0

    lax.fori_loop(0, SLEN // L, shift, 0, unroll=False)

    def row(r, _):
        i = i0 + r
        start = (Q - 1) - i
        ss = lax.bitwise_and(start, 7)
        q8 = lax.shift_right_logical(start, 3)
        src_off = ss * SLEN + q8 * 8
        dst_off = (h * Q + i) * K
        pltpu.async_copy(
            shift_v.at[pl.ds(src_off, K)], out_hbm.at[pl.ds(dst_off, K)], sem)

        @pl.when(r >= _WINDOW)
        def _():
            pltpu.make_async_copy(
                shift_v.at[pl.ds(0, K)], out_hbm.at[pl.ds(0, K)], sem).wait()

        return 0

    lax.fori_loop(0, Q // 2, row, 0, unroll=False)

    for _ in range(_WINDOW):
        pltpu.make_async_copy(
            shift_v.at[pl.ds(0, K)], out_hbm.at[pl.ds(0, K)], sem).wait()


@jax.jit
def _launch(weight, off_vec):
    mesh = plsc.VectorSubcoreMesh(core_axis_name="c", subcore_axis_name="s")
    f = functools.partial(
        pl.kernel,
        out_type=jax.ShapeDtypeStruct((H * Q * K,), jnp.float32),
        mesh=mesh,
        scratch_types=[
            pltpu.VMEM((NB, H), jnp.float32),
            pltpu.VMEM((L,), jnp.int32),
            pltpu.VMEM((RLEN,), jnp.float32),
            pltpu.VMEM((8 * SLEN,), jnp.float32),
            pltpu.SemaphoreType.DMA,
        ],
        compiler_params=pltpu.CompilerParams(needs_layout_passes=False),
    )(_body)
    return f(weight, off_vec)


def kernel(weight, q_len, k_len):
    off = jnp.full((L,), jnp.int32(k_len) - jnp.int32(q_len), dtype=jnp.int32)
    out = _launch(weight.astype(jnp.float32), off)
    return out.reshape(1, H, Q, K)


# speedup vs baseline: 91.7018x; 2.1985x over previous
import functools
import jax
import jax.numpy as jnp
from jax import lax
from jax.experimental import pallas as pl
from jax.experimental.pallas import tpu as pltpu, tpu_sc as plsc

H = 16
Q = 2048
K = 2048
NB = 32
L = 16
RLEN = 4128
NTILE = 31

_THRESH = (1, 2, 3, 4, 5, 6, 7, 8, 9, 10, 11, 12, 13, 14, 15, 16,
           19, 21, 24, 27, 31, 35, 40, 46, 52, 59, 67, 77, 87, 99, 113)


def _body(weight_hbm, off_hbm, out_hbm, w_v, off_v, rarr_v, s0_v, s1_v, sem0, sem1):
    c = lax.axis_index("c")
    s = lax.axis_index("s")
    h = s

    pltpu.sync_copy(weight_hbm, w_v)
    pltpu.sync_copy(off_hbm, off_v)

    off_vec = off_v[...]
    h_vec = jnp.full((L,), h, dtype=jnp.int32)
    iota = lax.iota(jnp.int32, L)

    def build_rarr(u, _):
        v = u * L + iota
        n = jnp.maximum((Q - 1) - off_vec - v, 0)
        b = jnp.zeros((L,), jnp.int32)
        for t in _THRESH:
            b = b + (n >= t).astype(jnp.int32)
        rarr_v[pl.ds(u * L, L)] = plsc.load_gather(w_v, [b, h_vec])
        return 0

    lax.fori_loop(0, RLEN // L, build_rarr, 0, unroll=False)

    def run_unit(g, s_v, sem):
        phi = 127 - 8 * g

        def build_tile(p, _):
            for r in range(8):
                base = p * 128 + (phi - r)
                for cb in range(8):
                    s_v[p, r, pl.ds(cb * L, L)] = plsc.load_gather(
                        rarr_v, [base + cb * L + iota])
            return 0

        lax.fori_loop(0, NTILE, build_tile, 0, unroll=False)

        def fire(k, _):
            b = g + 16 * k
            for t in range(16):
                pltpu.async_copy(
                    s_v.at[15 - k + t],
                    out_hbm.at[h, pl.ds(b * 8, 8), pl.ds(128 * t, 128)],
                    sem,
                )
            return 0

        lax.fori_loop(0, 16, fire, 0, unroll=False)

    def drain(s_v, sem):
        def d(_, __):
            pltpu.make_async_copy(
                s_v.at[0], out_hbm.at[0, pl.ds(0, 8), pl.ds(0, 128)], sem
            ).wait()
            return 0

        lax.fori_loop(0, 256, d, 0, unroll=False)

    bufs = ((s0_v, sem0), (s1_v, sem1))
    for j in range(8):
        s_v, sem = bufs[j % 2]
        if j >= 2:
            drain(s_v, sem)
        run_unit(c * 8 + j, s_v, sem)

    drain(*bufs[0])
    drain(*bufs[1])


@jax.jit
def _launch(weight, off_vec):
    mesh = plsc.VectorSubcoreMesh(core_axis_name="c", subcore_axis_name="s")
    f = functools.partial(
        pl.kernel,
        out_type=jax.ShapeDtypeStruct((H, Q, K), jnp.float32),
        mesh=mesh,
        scratch_types=[
            pltpu.VMEM((NB, H), jnp.float32),
            pltpu.VMEM((L,), jnp.int32),
            pltpu.VMEM((RLEN,), jnp.float32),
            pltpu.VMEM((NTILE, 8, 128), jnp.float32),
            pltpu.VMEM((NTILE, 8, 128), jnp.float32),
            pltpu.SemaphoreType.DMA,
            pltpu.SemaphoreType.DMA,
        ],
        compiler_params=pltpu.CompilerParams(needs_layout_passes=False),
    )(_body)
    return f(weight, off_vec)


def kernel(weight, q_len, k_len):
    off = jnp.full((L,), jnp.int32(k_len) - jnp.int32(q_len), dtype=jnp.int32)
    out = _launch(weight.astype(jnp.float32), off)
    return out[None]


# speedup vs baseline: 93.7466x; 1.0223x over previous
import functools
import jax
import jax.numpy as jnp
from jax import lax
from jax.experimental import pallas as pl
from jax.experimental.pallas import tpu as pltpu, tpu_sc as plsc

H = 16
Q = 2048
K = 2048
NB = 32
L = 16
RLEN = 4128
NTILE = 31

_THRESH = (1, 2, 3, 4, 5, 6, 7, 8, 9, 10, 11, 12, 13, 14, 15, 16,
           19, 21, 24, 27, 31, 35, 40, 46, 52, 59, 67, 77, 87, 99, 113)


def _body(weight_hbm, off_hbm, out_hbm, w_v, off_v, lut_v, rarr_v, s0_v, s1_v, sem0, sem1):
    c = lax.axis_index("c")
    s = lax.axis_index("s")
    h = s

    pltpu.sync_copy(weight_hbm, w_v)
    pltpu.sync_copy(off_hbm, off_v)

    off_vec = off_v[...]
    h_vec = jnp.full((L,), h, dtype=jnp.int32)
    iota = lax.iota(jnp.int32, L)

    def build_lut(u, _):
        n = u * L + iota
        b = jnp.zeros((L,), jnp.int32)
        for t in _THRESH:
            b = b + (n >= t).astype(jnp.int32)
        lut_v[pl.ds(u * L, L)] = plsc.load_gather(w_v, [b, h_vec])
        return 0

    lax.fori_loop(0, 128 // L, build_lut, 0, unroll=False)

    def build_rarr(u, _):
        v = u * L + iota
        n = jnp.minimum(jnp.maximum((Q - 1) - off_vec - v, 0), 127)
        rarr_v[pl.ds(u * L, L)] = plsc.load_gather(lut_v, [n])
        return 0

    lax.fori_loop(0, RLEN // L, build_rarr, 0, unroll=False)

    def run_unit(g, s_v, sem):
        phi = 127 - 8 * g

        def build_tile(p, _):
            for r in range(8):
                base = p * 128 + (phi - r)
                for cb in range(8):
                    s_v[p, r, pl.ds(cb * L, L)] = plsc.load_gather(
                        rarr_v, [base + cb * L + iota])
            return 0

        lax.fori_loop(0, NTILE, build_tile, 0, unroll=False)

        def fire(k, _):
            b = g + 16 * k
            for t in range(16):
                pltpu.async_copy(
                    s_v.at[15 - k + t],
                    out_hbm.at[h, pl.ds(b * 8, 8), pl.ds(128 * t, 128)],
                    sem,
                )
            return 0

        lax.fori_loop(0, 16, fire, 0, unroll=False)

    def drain(s_v, sem):
        def d(_, __):
            pltpu.make_async_copy(
                s_v.at[0], out_hbm.at[0, pl.ds(0, 8), pl.ds(0, 128)], sem
            ).wait()
            return 0

        lax.fori_loop(0, 256, d, 0, unroll=False)

    bufs = ((s0_v, sem0), (s1_v, sem1))
    for j in range(8):
        s_v, sem = bufs[j % 2]
        if j >= 2:
            drain(s_v, sem)
        run_unit(c * 8 + j, s_v, sem)

    drain(*bufs[0])
    drain(*bufs[1])


@jax.jit
def _launch(weight, off_vec):
    mesh = plsc.VectorSubcoreMesh(core_axis_name="c", subcore_axis_name="s")
    f = functools.partial(
        pl.kernel,
        out_type=jax.ShapeDtypeStruct((H, Q, K), jnp.float32),
        mesh=mesh,
        scratch_types=[
            pltpu.VMEM((NB, H), jnp.float32),
            pltpu.VMEM((L,), jnp.int32),
            pltpu.VMEM((128,), jnp.float32),
            pltpu.VMEM((RLEN,), jnp.float32),
            pltpu.VMEM((NTILE, 8, 128), jnp.float32),
            pltpu.VMEM((NTILE, 8, 128), jnp.float32),
            pltpu.SemaphoreType.DMA,
            pltpu.SemaphoreType.DMA,
        ],
        compiler_params=pltpu.CompilerParams(needs_layout_passes=False),
    )(_body)
    return f(weight, off_vec)


def kernel(weight, q_len, k_len):
    off = jnp.full((L,), jnp.int32(k_len) - jnp.int32(q_len), dtype=jnp.int32)
    out = _launch(weight.astype(jnp.float32), off)
    return out[None]


# speedup vs baseline: 132.6672x; 1.4152x over previous
import functools
import jax
import jax.numpy as jnp
from jax import lax
from jax.experimental import pallas as pl
from jax.experimental.pallas import tpu as pltpu, tpu_sc as plsc

H = 16
Q = 2048
K = 2048
NB = 32
L = 16
RLEN = 4128
NTILE = 31

_THRESH = (1, 2, 3, 4, 5, 6, 7, 8, 9, 10, 11, 12, 13, 14, 15, 16,
           19, 21, 24, 27, 31, 35, 40, 46, 52, 59, 67, 77, 87, 99, 113)


def _body(weight_hbm, off_hbm, out_hbm, w_v, off_v, lut_v, rarr_v, s0_v, s1_v, sem0, sem1):
    c = lax.axis_index("c")
    s = lax.axis_index("s")
    h = s

    pltpu.sync_copy(weight_hbm, w_v)
    pltpu.sync_copy(off_hbm, off_v)

    off_vec = off_v[...]
    h_vec = jnp.full((L,), h, dtype=jnp.int32)
    iota = lax.iota(jnp.int32, L)

    def build_lut(u, _):
        n = u * L + iota
        b = jnp.zeros((L,), jnp.int32)
        for t in _THRESH:
            b = b + (n >= t).astype(jnp.int32)
        lut_v[pl.ds(u * L, L)] = plsc.load_gather(w_v, [b, h_vec])
        return 0

    lax.fori_loop(0, 128 // L, build_lut, 0, unroll=False)

    def build_rarr(u, _):
        v = u * L + iota
        n = jnp.minimum(jnp.maximum((Q - 1) - off_vec - v, 0), 127)
        rarr_v[pl.ds(u * L, L)] = plsc.load_gather(lut_v, [n])
        return 0

    lax.fori_loop(0, RLEN // L, build_rarr, 0, unroll=False)

    def run_unit(g, s_v, sem):
        phi = 127 - 8 * g

        @plsc.parallel_loop(0, NTILE)
        def build_tile(p):
            for r in range(8):
                base = p * 128 + (phi - r)
                for cb in range(8):
                    s_v[p, r, pl.ds(cb * L, L)] = plsc.load_gather(
                        rarr_v, [base + cb * L + iota])

        def fire(k, _):
            b = g + 16 * k
            for t in range(16):
                pltpu.async_copy(
                    s_v.at[15 - k + t],
                    out_hbm.at[h, pl.ds(b * 8, 8), pl.ds(128 * t, 128)],
                    sem,
                )
            return 0

        lax.fori_loop(0, 16, fire, 0, unroll=False)

    def drain(s_v, sem):
        def d(_, __):
            pltpu.make_async_copy(
                s_v.at[0], out_hbm.at[0, pl.ds(0, 8), pl.ds(0, 128)], sem
            ).wait()
            return 0

        lax.fori_loop(0, 256, d, 0, unroll=False)

    bufs = ((s0_v, sem0), (s1_v, sem1))
    for j in range(8):
        s_v, sem = bufs[j % 2]
        if j >= 2:
            drain(s_v, sem)
        run_unit(c * 8 + j, s_v, sem)

    drain(*bufs[0])
    drain(*bufs[1])


@jax.jit
def _launch(weight, off_vec):
    mesh = plsc.VectorSubcoreMesh(core_axis_name="c", subcore_axis_name="s")
    f = functools.partial(
        pl.kernel,
        out_type=jax.ShapeDtypeStruct((H, Q, K), jnp.float32),
        mesh=mesh,
        scratch_types=[
            pltpu.VMEM((NB, H), jnp.float32),
            pltpu.VMEM((L,), jnp.int32),
            pltpu.VMEM((128,), jnp.float32),
            pltpu.VMEM((RLEN,), jnp.float32),
            pltpu.VMEM((NTILE, 8, 128), jnp.float32),
            pltpu.VMEM((NTILE, 8, 128), jnp.float32),
            pltpu.SemaphoreType.DMA,
            pltpu.SemaphoreType.DMA,
        ],
        compiler_params=pltpu.CompilerParams(needs_layout_passes=False),
    )(_body)
    return f(weight, off_vec)


def kernel(weight, q_len, k_len):
    off = jnp.full((L,), jnp.int32(k_len) - jnp.int32(q_len), dtype=jnp.int32)
    out = _launch(weight.astype(jnp.float32), off)
    return out[None]


# speedup vs baseline: 136.0176x; 1.0253x over previous
import functools
import jax
import jax.numpy as jnp
from jax import lax
from jax.experimental import pallas as pl
from jax.experimental.pallas import tpu as pltpu, tpu_sc as plsc

H = 16
Q = 2048
K = 2048
NB = 32
L = 16
RLEN = 4128
NTILE = 31

_THRESH = (1, 2, 3, 4, 5, 6, 7, 8, 9, 10, 11, 12, 13, 14, 15, 16,
           19, 21, 24, 27, 31, 35, 40, 46, 52, 59, 67, 77, 87, 99, 113)


def _body(weight_hbm, off_hbm, out_hbm, w_v, off_v, lut_v, rarr_v, s0_v, s1_v, sem0, sem1):
    c = lax.axis_index("c")
    s = lax.axis_index("s")
    h = s

    pltpu.sync_copy(weight_hbm, w_v)
    pltpu.sync_copy(off_hbm, off_v)

    off_vec = off_v[...]
    h_vec = jnp.full((L,), h, dtype=jnp.int32)
    iota = lax.iota(jnp.int32, L)

    def build_lut(u, _):
        n = u * L + iota
        b = jnp.zeros((L,), jnp.int32)
        for t in _THRESH:
            b = b + (n >= t).astype(jnp.int32)
        lut_v[pl.ds(u * L, L)] = plsc.load_gather(w_v, [b, h_vec])
        return 0

    lax.fori_loop(0, 128 // L, build_lut, 0, unroll=False)

    def build_rarr(u, _):
        v = u * L + iota
        n = jnp.minimum(jnp.maximum((Q - 1) - off_vec - v, 0), 127)
        rarr_v[pl.ds(u * L, L)] = plsc.load_gather(lut_v, [n])
        return 0

    lax.fori_loop(0, RLEN // L, build_rarr, 0, unroll=False)

    def run_unit(g, s_v, sem):
        phi = 127 - 8 * g

        @plsc.parallel_loop(0, NTILE)
        def build_tile(p):
            for r in range(8):
                base = p * 128 + (phi - r)
                for cb in range(8):
                    s_v[p, r, pl.ds(cb * L, L)] = plsc.load_gather(
                        rarr_v, [base + cb * L + iota])

        @plsc.parallel_loop(0, 16)
        def fire(k):
            b = g + 16 * k
            for t in range(16):
                pltpu.async_copy(
                    s_v.at[15 - k + t],
                    out_hbm.at[h, pl.ds(b * 8, 8), pl.ds(128 * t, 128)],
                    sem,
                )

    def drain(s_v, sem):
        def d(_, __):
            for _u in range(4):
                pltpu.make_async_copy(
                    s_v.at[0], out_hbm.at[0, pl.ds(0, 8), pl.ds(0, 128)], sem
                ).wait()
            return 0

        lax.fori_loop(0, 64, d, 0, unroll=False)

    bufs = ((s0_v, sem0), (s1_v, sem1))
    for j in range(8):
        s_v, sem = bufs[j % 2]
        if j >= 2:
            drain(s_v, sem)
        run_unit(c * 8 + j, s_v, sem)

    drain(*bufs[0])
    drain(*bufs[1])


@jax.jit
def _launch(weight, off_vec):
    mesh = plsc.VectorSubcoreMesh(core_axis_name="c", subcore_axis_name="s")
    f = functools.partial(
        pl.kernel,
        out_type=jax.ShapeDtypeStruct((H, Q, K), jnp.float32),
        mesh=mesh,
        scratch_types=[
            pltpu.VMEM((NB, H), jnp.float32),
            pltpu.VMEM((L,), jnp.int32),
            pltpu.VMEM((128,), jnp.float32),
            pltpu.VMEM((RLEN,), jnp.float32),
            pltpu.VMEM((NTILE, 8, 128), jnp.float32),
            pltpu.VMEM((NTILE, 8, 128), jnp.float32),
            pltpu.SemaphoreType.DMA,
            pltpu.SemaphoreType.DMA,
        ],
        compiler_params=pltpu.CompilerParams(needs_layout_passes=False),
    )(_body)
    return f(weight, off_vec)


def kernel(weight, q_len, k_len):
    off = jnp.full((L,), jnp.int32(k_len) - jnp.int32(q_len), dtype=jnp.int32)
    out = _launch(weight.astype(jnp.float32), off)
    return out[None]
